# Initial kernel scaffold; baseline (speedup 1.0000x reference)
import jax
import jax.numpy as jnp
from jax.experimental import pallas as pl


def kernel(v, l, e, e_mask, vbi, vfc, vfb, W_feat, b_feat, W_lab, b_lab, W_msg, b_msg, W_upd, b_upd):
    raise NotImplementedError("write your pallas kernel here")


# speedup vs baseline: 7.0336x; 7.0336x over previous
import functools

import jax
import jax.numpy as jnp
import numpy as np
from jax import lax
from jax.experimental import pallas as pl
from jax.experimental.pallas import tpu as pltpu
from jax.experimental.pallas import tpu_sc as plsc

N = 50000
E = 800000
H = 32
D = 64
B = 16

NC = 2
NS = 16
LANES = 16

NPAD = 50048
EPAD = 819200
NR = EPAD // 128
RPT = NR // NS
RPW = NR // (NC * NS)
SUP = 8
HSUP = SUP // 2
NSUP = RPT // SUP
SUPW = RPW // SUP
NPT = NPAD // NS

BN = 2000
NB = N // BN


def _pe_table() -> np.ndarray:
    pe = np.zeros((20, H), dtype=np.float32)
    position = np.arange(0, 20, dtype=np.float32)[:, None]
    div_term = np.exp(np.arange(0, H, 2, dtype=np.float32) * (-np.log(10000.0) / H))
    pe[:, 0::2] = np.sin(position * div_term)
    pe[:, 1::2] = np.cos(position * div_term)
    return np.concatenate([pe, np.zeros((4, H), np.float32)], axis=0)


_BIGI = np.int32(2**30)


def _pool_body(vfc_ref, vbi_ref, out_ref):
    vfc = vfc_ref[...]
    vbi = vbi_ref[...]
    acc = jnp.zeros((B, 128), jnp.float32)
    sub = lax.broadcasted_iota(jnp.int32, (B, 128), 0)
    for b in range(B):
        mb = jnp.min(jnp.where(vbi == b, vfc, _BIGI))
        acc = jnp.where(sub == b, mb.astype(jnp.float32), acc)
    out_ref[...] = acc


def _encode_body(v_ref, l_ref, vfc_ref, vbi_ref, pool_ref, pe_ref, wf_ref,
                 bf_ref, wl_ref, bl_ref, wm3_ref, wm2_ref,
                 x_ref, pw3_ref, g_ref):
    f32 = jnp.float32
    vb = v_ref[...]
    z2 = jnp.zeros((BN, 2), f32)
    nonpos = jnp.concatenate([vb[:, 0:3], vb[:, 6:9], z2], axis=-1)
    pos8 = jnp.concatenate([vb[:, 3:6], jnp.zeros((BN, 5), f32)], axis=-1)
    h = jnp.dot(nonpos, wf_ref[...], precision="highest") + bf_ref[0:1, :]
    pw3 = jnp.dot(pos8, wm3_ref[...], precision="highest")
    vbi = vbi_ref[...]
    vfc = vfc_ref[...]
    oh16 = (vbi == lax.broadcasted_iota(jnp.int32, (BN, B), 1)).astype(f32)
    poolg = jnp.dot(oh16, pool_ref[...], precision="highest")[:, 0:1]
    lvl = vfc - poolg.astype(jnp.int32)
    oh24 = (lvl == lax.broadcasted_iota(jnp.int32, (BN, 24), 1)).astype(f32)
    pe_emb = jnp.dot(oh24, pe_ref[...], precision="highest")
    le = jnp.dot(l_ref[...], wl_ref[...], precision="highest") + bl_ref[0:1, :]
    x = jnp.concatenate([h + pe_emb, le], axis=-1)
    x_ref[...] = x
    pw3_ref[...] = pw3
    g = jnp.dot(x, wm2_ref[...], precision="highest") - pw3
    g_ref[0] = g[:, :H]
    g_ref[1] = g[:, H:]


def _update_body(emit_g, x_ref, s_ref, pw3_ref, dw_ref, wm1_ref, bm_ref,
                 wu1_ref, wu2_ref, bu_ref, wm2_ref, xo_ref, *maybe_g):
    xb = x_ref[...]
    pw3 = pw3_ref[...]
    sb = jnp.concatenate([s_ref[0], s_ref[1]], axis=-1)
    dwb = dw_ref[...]
    deg = jnp.maximum(dwb[:, 0:1] + dwb[:, 2:3], 1.0)
    wsum = dwb[:, 1:2] + dwb[:, 3:4]
    base = jnp.dot(xb, wm1_ref[...], precision="highest") + pw3 + bm_ref[0:1, :]
    aggr = (base * wsum + sb) / deg
    xn = (xb + jnp.dot(xb, wu1_ref[...], precision="highest")
          + jnp.dot(aggr, wu2_ref[...], precision="highest") + bu_ref[0:1, :])
    xo_ref[...] = xn
    if emit_g:
        g_ref = maybe_g[0]
        g = jnp.dot(xn, wm2_ref[...], precision="highest") - pw3
        g_ref[0] = g[:, :H]
        g_ref[1] = g[:, H:]


def _row_spec(w):
    return pl.BlockSpec((BN, w), lambda i: (i, 0))


def _full_spec(shape):
    nd = len(shape)
    return pl.BlockSpec(shape, lambda i, _n=nd: (0,) * _n)


def _split_spec(w):
    return pl.BlockSpec((2, BN, w), lambda i: (0, i, 0))


@functools.cache
def _pool_call():
    return pl.pallas_call(
        _pool_body,
        out_shape=jax.ShapeDtypeStruct((B, 128), jnp.float32),
    )


@functools.cache
def _encode_call():
    return pl.pallas_call(
        _encode_body,
        grid=(NB,),
        in_specs=[
            _row_spec(9), _row_spec(8), _row_spec(1), _row_spec(1),
            _full_spec((B, 128)), _full_spec((24, H)), _full_spec((8, H)),
            _full_spec((8, H)), _full_spec((8, H)), _full_spec((8, H)),
            _full_spec((8, D)), _full_spec((D, D)),
        ],
        out_specs=[_row_spec(D), _row_spec(D), _split_spec(H)],
        out_shape=[
            jax.ShapeDtypeStruct((N, D), jnp.float32),
            jax.ShapeDtypeStruct((N, D), jnp.float32),
            jax.ShapeDtypeStruct((2, NPAD, H), jnp.float32),
        ],
    )


@functools.cache
def _update_call(emit_g):
    out_specs = [_row_spec(D)]
    out_shape = [jax.ShapeDtypeStruct((N, D), jnp.float32)]
    if emit_g:
        out_specs.append(_split_spec(H))
        out_shape.append(jax.ShapeDtypeStruct((2, NPAD, H), jnp.float32))
    return pl.pallas_call(
        functools.partial(_update_body, emit_g),
        grid=(NB,),
        in_specs=[
            _row_spec(D), _split_spec(H), _row_spec(D),
            pl.BlockSpec((BN, 4), lambda i: (i, 0)),
            _full_spec((D, D)), _full_spec((8, D)), _full_spec((D, D)),
            _full_spec((D, D)), _full_spec((8, D)), _full_spec((D, D)),
        ],
        out_specs=out_specs,
        out_shape=out_shape,
    )


def _scatter_rows_body(sidx_ref, dst_ref, mask_ref, g2_ref, out_ref,
                       idxbuf, dstbuf, mbuf, rows, s_sh, semg, sems):
    c = lax.axis_index("c")
    s = lax.axis_index("s")

    @pl.loop(0, HSUP * 128)
    def _zero(i):
        z = jnp.zeros((LANES,), jnp.float32)
        rows[i, pl.ds(0, LANES)] = z
        rows[i, pl.ds(LANES, LANES)] = z

    for t in range(6):
        pltpu.sync_copy(rows, s_sh.at[pl.ds(s * NPT + t * 512, 512), :])
    pltpu.sync_copy(rows.at[pl.ds(0, NPT - 6 * 512), :],
                    s_sh.at[pl.ds(s * NPT + 6 * 512, NPT - 6 * 512), :])
    plsc.subcore_barrier()

    @pl.loop(0, NSUP)
    def _super(sup):
        row0 = s * RPT + sup * SUP
        pltpu.sync_copy(sidx_ref.at[c, pl.ds(row0, SUP), :], idxbuf)
        pltpu.sync_copy(dst_ref.at[pl.ds(row0, SUP), :], dstbuf)
        pltpu.sync_copy(mask_ref.at[pl.ds(row0 * 128, SUP * 128)], mbuf)
        for half in range(2):
            gathers = []
            for b in range(HSUP):
                gathers.append(pltpu.async_copy(
                    g2_ref.at[idxbuf.at[half * HSUP + b]],
                    rows.at[pl.ds(b * 128, 128), :], semg))
            for gd in gathers:
                gd.wait()

            @pl.loop(0, HSUP * 128 // LANES)
            def _group(gi):
                mv = mbuf[pl.ds(half * HSUP * 128 + gi * LANES, LANES)]
                for j in range(LANES):
                    k = gi * LANES + j
                    mj = jnp.full((LANES,), mv[j], jnp.float32)
                    rows[k, pl.ds(0, LANES)] = rows[k, pl.ds(0, LANES)] * mj
                    rows[k, pl.ds(LANES, LANES)] = rows[k, pl.ds(LANES, LANES)] * mj

            scatters = []
            for b in range(HSUP):
                scatters.append(pltpu.async_copy(
                    rows.at[pl.ds(b * 128, 128), :],
                    s_sh.at[dstbuf.at[half * HSUP + b]], sems, add=True))
            for sd in scatters:
                sd.wait()

    plsc.subcore_barrier()
    for t in range(6):
        pltpu.sync_copy(s_sh.at[pl.ds(s * NPT + t * 512, 512), :], rows)
        pltpu.sync_copy(rows, out_ref.at[pl.ds(c * NPAD + s * NPT + t * 512, 512), :])
    tail = NPT - 6 * 512
    pltpu.sync_copy(s_sh.at[pl.ds(s * NPT + 6 * 512, tail), :],
                    rows.at[pl.ds(0, tail), :])
    pltpu.sync_copy(rows.at[pl.ds(0, tail), :],
                    out_ref.at[pl.ds(c * NPAD + s * NPT + 6 * 512, tail), :])


@functools.cache
def _scatter_rows_call():
    mesh = plsc.VectorSubcoreMesh(core_axis_name="c", subcore_axis_name="s")
    return pl.kernel(
        _scatter_rows_body,
        out_type=jax.ShapeDtypeStruct((2 * NPAD, H), jnp.float32),
        mesh=mesh,
        scratch_types=[
            pltpu.VMEM((SUP, 128), jnp.int32),
            pltpu.VMEM((SUP, 128), jnp.int32),
            pltpu.VMEM((SUP * 128,), jnp.float32),
            pltpu.VMEM((HSUP * 128, H), jnp.float32),
            pltpu.VMEM_SHARED((NPAD, H), jnp.float32),
            pltpu.SemaphoreType.DMA,
            pltpu.SemaphoreType.DMA,
        ],
        compiler_params=pltpu.CompilerParams(use_tc_tiling_on_sc=False),
    )


_ZPT = 2 * NPAD // NS


def _degwsum_body(dst_ref, dstw_ref, mask_ref, ones_ref, out_ref,
                  dstbuf, dstwbuf, mbuf, obuf, dw_sh, sems):
    c = lax.axis_index("c")
    s = lax.axis_index("s")
    wid = c * NS + s

    @pl.loop(0, SUP * 128 // LANES)
    def _zero(i):
        obuf[pl.ds(i * LANES, LANES)] = jnp.zeros((LANES,), jnp.float32)

    for t in range(6):
        pltpu.sync_copy(obuf, dw_sh.at[pl.ds(s * _ZPT + t * 1024, 1024)])
    pltpu.sync_copy(obuf.at[pl.ds(0, _ZPT - 6 * 1024)],
                    dw_sh.at[pl.ds(s * _ZPT + 6 * 1024, _ZPT - 6 * 1024)])
    plsc.subcore_barrier()

    @pl.loop(0, SUPW)
    def _chunk(ch):
        row0 = wid * RPW + ch * SUP
        pltpu.sync_copy(dst_ref.at[pl.ds(row0, SUP), :], dstbuf)
        pltpu.sync_copy(dstw_ref.at[pl.ds(row0, SUP), :], dstwbuf)
        pltpu.sync_copy(mask_ref.at[pl.ds(row0 * 128, SUP * 128)], mbuf)
        pltpu.sync_copy(ones_ref.at[pl.ds(row0 * 128, SUP * 128)], obuf)
        copies = []
        for b in range(SUP):
            copies.append(pltpu.async_copy(
                obuf.at[pl.ds(b * 128, 128)], dw_sh.at[dstbuf.at[b]],
                sems, add=True))
            copies.append(pltpu.async_copy(
                mbuf.at[pl.ds(b * 128, 128)], dw_sh.at[dstwbuf.at[b]],
                sems, add=True))
        for cd in copies:
            cd.wait()

    plsc.subcore_barrier()
    for t in range(6):
        pltpu.sync_copy(dw_sh.at[pl.ds(s * _ZPT + t * 1024, 1024)], obuf)
        pltpu.sync_copy(obuf, out_ref.at[pl.ds(c * 2 * NPAD + s * _ZPT + t * 1024, 1024)])
    tail = _ZPT - 6 * 1024
    pltpu.sync_copy(dw_sh.at[pl.ds(s * _ZPT + 6 * 1024, tail)], obuf.at[pl.ds(0, tail)])
    pltpu.sync_copy(obuf.at[pl.ds(0, tail)],
                    out_ref.at[pl.ds(c * 2 * NPAD + s * _ZPT + 6 * 1024, tail)])


@functools.cache
def _degwsum_call():
    mesh = plsc.VectorSubcoreMesh(core_axis_name="c", subcore_axis_name="s")
    return pl.kernel(
        _degwsum_body,
        out_type=jax.ShapeDtypeStruct((4 * NPAD,), jnp.float32),
        mesh=mesh,
        scratch_types=[
            pltpu.VMEM((SUP, 128), jnp.int32),
            pltpu.VMEM((SUP, 128), jnp.int32),
            pltpu.VMEM((SUP * 128,), jnp.float32),
            pltpu.VMEM((SUP * 128,), jnp.float32),
            pltpu.VMEM_SHARED((2 * NPAD,), jnp.float32),
            pltpu.SemaphoreType.DMA,
        ],
        compiler_params=pltpu.CompilerParams(use_tc_tiling_on_sc=False),
    )


def kernel(v, l, e, e_mask, vbi, vfc, vfb, W_feat, b_feat, W_lab, b_lab,
           W_msg, b_msg, W_upd, b_upd):
    f32 = jnp.float32
    i32 = jnp.int32

    src = e[0].astype(i32)
    dst = e[1].astype(i32)
    pad = EPAD - E
    srcp = jnp.concatenate([src, jnp.zeros((pad,), i32)])
    dstp = jnp.concatenate([dst, jnp.zeros((pad,), i32)])
    maskp = jnp.concatenate([e_mask.astype(f32), jnp.zeros((pad,), f32)])
    sidx = jnp.stack([srcp, srcp + NPAD]).reshape(2, NR, 128)
    dst2 = dstp.reshape(NR, 128)
    dstw2 = (dstp + NPAD).reshape(NR, 128)
    onesp = jnp.concatenate([jnp.ones((E,), f32), jnp.zeros((pad,), f32)])

    npad = 51200 - N
    vfcp = jnp.concatenate([vfc.astype(i32), jnp.full((npad,), _BIGI, i32)]).reshape(400, 128)
    vbip = jnp.concatenate([vbi.astype(i32), jnp.full((npad,), -1, i32)]).reshape(400, 128)

    pe24 = jnp.asarray(_pe_table())
    wf8 = jnp.concatenate([W_feat, jnp.zeros((2, H), f32)], axis=0)
    bf8 = jnp.broadcast_to(b_feat, (8, H))
    bl8 = jnp.broadcast_to(b_lab, (8, H))
    wm1 = W_msg[:D]
    wm2 = W_msg[D:2 * D]
    wm3 = jnp.concatenate([W_msg[2 * D:], jnp.zeros((5, D), f32)], axis=0)
    bm8 = jnp.broadcast_to(b_msg, (8, D))
    wu1 = W_upd[:D]
    wu2 = W_upd[D:]
    bu8 = jnp.broadcast_to(b_upd, (8, D))

    pool = _pool_call()(vfcp, vbip)
    x, pw3, g = _encode_call()(
        v, l, vfc.reshape(N, 1).astype(i32), vbi.reshape(N, 1).astype(i32),
        pool, pe24, wf8, bf8, W_lab, bl8, wm3, wm2)
    dw = jnp.transpose(_degwsum_call()(dst2, dstw2, maskp, onesp).reshape(4, NPAD))

    upd = _update_call(True)
    upd_last = _update_call(False)
    for layer in range(3):
        s2 = _scatter_rows_call()(sidx, dst2, maskp, g.reshape(2 * NPAD, H))
        s3 = s2.reshape(2, NPAD, H)
        if layer < 2:
            x, g = upd(x, s3, pw3, dw, wm1, bm8, wu1, wu2, bu8, wm2)
        else:
            (x,) = upd_last(x, s3, pw3, dw, wm1, bm8, wu1, wu2, bu8, wm2)
    return x


# speedup vs baseline: 9.3932x; 1.3355x over previous
import functools

import jax
import jax.numpy as jnp
import numpy as np
from jax import lax
from jax.experimental import pallas as pl
from jax.experimental.pallas import tpu as pltpu
from jax.experimental.pallas import tpu_sc as plsc

N = 50000
E = 800000
H = 32
D = 64
B = 16

NC = 2
NS = 16
LANES = 16

NPAD = 50048
EPAD = 811008
NR = EPAD // 128
RPT = NR // NS
RPW = NR // (NC * NS)
SUP = 6
HSUP = SUP // 2
CHE = SUP * 128
HE = CHE // 2
NSUP = RPT // SUP
SUPW = RPW // SUP
NPT = NPAD // NS

BN = 2000
NB = N // BN


def _pe_table() -> np.ndarray:
    pe = np.zeros((20, H), dtype=np.float32)
    position = np.arange(0, 20, dtype=np.float32)[:, None]
    div_term = np.exp(np.arange(0, H, 2, dtype=np.float32) * (-np.log(10000.0) / H))
    pe[:, 0::2] = np.sin(position * div_term)
    pe[:, 1::2] = np.cos(position * div_term)
    return np.concatenate([pe, np.zeros((4, H), np.float32)], axis=0)


_BIGI = np.int32(2**30)


def _pool_body(vfc_ref, vbi_ref, out_ref):
    vfc = vfc_ref[...]
    vbi = vbi_ref[...]
    acc = jnp.zeros((B, 128), jnp.float32)
    sub = lax.broadcasted_iota(jnp.int32, (B, 128), 0)
    for b in range(B):
        mb = jnp.min(jnp.where(vbi == b, vfc, _BIGI))
        acc = jnp.where(sub == b, mb.astype(jnp.float32), acc)
    out_ref[...] = acc


def _encode_body(v_ref, l_ref, vfc_ref, vbi_ref, pool_ref, pe_ref, wf_ref,
                 bf_ref, wl_ref, bl_ref, wm3_ref, wm2_ref,
                 x_ref, pw3_ref, g_ref):
    f32 = jnp.float32
    vb = v_ref[...]
    z2 = jnp.zeros((BN, 2), f32)
    nonpos = jnp.concatenate([vb[:, 0:3], vb[:, 6:9], z2], axis=-1)
    pos8 = jnp.concatenate([vb[:, 3:6], jnp.zeros((BN, 5), f32)], axis=-1)
    h = jnp.dot(nonpos, wf_ref[...], precision="highest") + bf_ref[0:1, :]
    pw3 = jnp.dot(pos8, wm3_ref[...], precision="highest")
    vbi = vbi_ref[...]
    vfc = vfc_ref[...]
    oh16 = (vbi == lax.broadcasted_iota(jnp.int32, (BN, B), 1)).astype(f32)
    poolg = jnp.dot(oh16, pool_ref[...], precision="highest")[:, 0:1]
    lvl = vfc - poolg.astype(jnp.int32)
    oh24 = (lvl == lax.broadcasted_iota(jnp.int32, (BN, 24), 1)).astype(f32)
    pe_emb = jnp.dot(oh24, pe_ref[...], precision="highest")
    le = jnp.dot(l_ref[...], wl_ref[...], precision="highest") + bl_ref[0:1, :]
    x = jnp.concatenate([h + pe_emb, le], axis=-1)
    x_ref[...] = x
    pw3_ref[...] = pw3
    g = jnp.dot(x, wm2_ref[...], precision="highest") - pw3
    g_ref[0] = g[:, :H]
    g_ref[1] = g[:, H:]


def _update_body(emit_g, x_ref, s_ref, pw3_ref, dw_ref, wm1_ref, bm_ref,
                 wu1_ref, wu2_ref, bu_ref, wm2_ref, xo_ref, *maybe_g):
    xb = x_ref[...]
    pw3 = pw3_ref[...]
    sb = jnp.concatenate([s_ref[0], s_ref[1]], axis=-1)
    dwb = dw_ref[...]
    deg = jnp.maximum(dwb[:, 0:1] + dwb[:, 2:3], 1.0)
    wsum = dwb[:, 1:2] + dwb[:, 3:4]
    base = jnp.dot(xb, wm1_ref[...], precision="highest") + pw3 + bm_ref[0:1, :]
    aggr = (base * wsum + sb) / deg
    xn = (xb + jnp.dot(xb, wu1_ref[...], precision="highest")
          + jnp.dot(aggr, wu2_ref[...], precision="highest") + bu_ref[0:1, :])
    xo_ref[...] = xn
    if emit_g:
        g_ref = maybe_g[0]
        g = jnp.dot(xn, wm2_ref[...], precision="highest") - pw3
        g_ref[0] = g[:, :H]
        g_ref[1] = g[:, H:]


def _row_spec(w):
    return pl.BlockSpec((BN, w), lambda i: (i, 0))


def _full_spec(shape):
    nd = len(shape)
    return pl.BlockSpec(shape, lambda i, _n=nd: (0,) * _n)


def _split_spec(w):
    return pl.BlockSpec((2, BN, w), lambda i: (0, i, 0))


@functools.cache
def _pool_call():
    return pl.pallas_call(
        _pool_body,
        out_shape=jax.ShapeDtypeStruct((B, 128), jnp.float32),
    )


@functools.cache
def _encode_call():
    return pl.pallas_call(
        _encode_body,
        grid=(NB,),
        in_specs=[
            _row_spec(9), _row_spec(8), _row_spec(1), _row_spec(1),
            _full_spec((B, 128)), _full_spec((24, H)), _full_spec((8, H)),
            _full_spec((8, H)), _full_spec((8, H)), _full_spec((8, H)),
            _full_spec((8, D)), _full_spec((D, D)),
        ],
        out_specs=[_row_spec(D), _row_spec(D), _split_spec(H)],
        out_shape=[
            jax.ShapeDtypeStruct((N, D), jnp.float32),
            jax.ShapeDtypeStruct((N, D), jnp.float32),
            jax.ShapeDtypeStruct((2, NPAD, H), jnp.float32),
        ],
    )


@functools.cache
def _update_call(emit_g):
    out_specs = [_row_spec(D)]
    out_shape = [jax.ShapeDtypeStruct((N, D), jnp.float32)]
    if emit_g:
        out_specs.append(_split_spec(H))
        out_shape.append(jax.ShapeDtypeStruct((2, NPAD, H), jnp.float32))
    return pl.pallas_call(
        functools.partial(_update_body, emit_g),
        grid=(NB,),
        in_specs=[
            _row_spec(D), _split_spec(H), _row_spec(D),
            pl.BlockSpec((BN, 4), lambda i: (i, 0)),
            _full_spec((D, D)), _full_spec((8, D)), _full_spec((D, D)),
            _full_spec((D, D)), _full_spec((8, D)), _full_spec((D, D)),
        ],
        out_specs=out_specs,
        out_shape=out_shape,
    )


def _scatter_rows_body(sidx_ref, dst_ref, mask_ref, g2_ref, out_ref,
                       idxbuf, dstbuf, mbuf, rows, s_sh,
                       semi, semga, semgb, semsa, semsb):
    c = lax.axis_index("c")
    s = lax.axis_index("s")

    @pl.loop(0, CHE)
    def _zero(i):
        z = jnp.zeros((LANES,), jnp.float32)
        rows[i, pl.ds(0, LANES)] = z
        rows[i, pl.ds(LANES, LANES)] = z

    for t in range(4):
        pltpu.sync_copy(rows, s_sh.at[pl.ds(s * NPT + t * CHE, CHE), :])
    pltpu.sync_copy(rows.at[pl.ds(0, NPT - 4 * CHE), :],
                    s_sh.at[pl.ds(s * NPT + 4 * CHE, NPT - 4 * CHE), :])

    obase = c * NPAD + s * NPT
    pltpu.async_copy(rows.at[pl.ds(0, HE), :], out_ref.at[pl.ds(obase, HE), :], semsa)
    pltpu.async_copy(rows.at[pl.ds(HE, HE), :], out_ref.at[pl.ds(obase + HE, HE), :], semsb)
    row00 = s * RPT
    pltpu.async_copy(sidx_ref.at[c, pl.ds(row00, SUP), :], idxbuf.at[0], semi)
    pltpu.async_copy(dst_ref.at[pl.ds(row00, SUP), :], dstbuf.at[0], semi)
    pltpu.async_copy(mask_ref.at[pl.ds(row00 * 128, CHE)], mbuf.at[0], semi)
    plsc.subcore_barrier()

    @pl.loop(0, NSUP)
    def _super(sup):
        p = lax.rem(sup, 2)
        q = 1 - p
        pltpu.make_async_copy(out_ref.at[pl.ds(obase, HE), :],
                              rows.at[pl.ds(0, HE), :], semsa).wait()
        pltpu.make_async_copy(out_ref.at[pl.ds(obase, HE), :],
                              rows.at[pl.ds(HE, HE), :], semsb).wait()
        pltpu.make_async_copy(sidx_ref.at[c, pl.ds(row00, SUP), :],
                              idxbuf.at[p], semi).wait()
        pltpu.make_async_copy(dst_ref.at[pl.ds(row00, SUP), :],
                              dstbuf.at[p], semi).wait()
        pltpu.make_async_copy(mask_ref.at[pl.ds(row00 * 128, CHE)],
                              mbuf.at[p], semi).wait()
        nrow0 = s * RPT + jnp.minimum(sup + 1, NSUP - 1) * SUP
        pltpu.async_copy(sidx_ref.at[c, pl.ds(nrow0, SUP), :], idxbuf.at[q], semi)
        pltpu.async_copy(dst_ref.at[pl.ds(nrow0, SUP), :], dstbuf.at[q], semi)
        pltpu.async_copy(mask_ref.at[pl.ds(nrow0 * 128, CHE)], mbuf.at[q], semi)
        ga, gb = [], []
        for b in range(HSUP):
            ga.append(pltpu.async_copy(
                g2_ref.at[idxbuf.at[p, b]],
                rows.at[pl.ds(b * 128, 128), :], semga))
        for b in range(HSUP):
            gb.append(pltpu.async_copy(
                g2_ref.at[idxbuf.at[p, HSUP + b]],
                rows.at[pl.ds(HE + b * 128, 128), :], semgb))
        for half, gds, sem_s in ((0, ga, semsa), (1, gb, semsb)):
            for gd in gds:
                gd.wait()

            @pl.loop(0, HE // LANES)
            def _group(gi):
                mv = mbuf[p, pl.ds(half * HE + gi * LANES, LANES)]
                for j in range(LANES):
                    k = half * HE + gi * LANES + j
                    mj = jnp.full((LANES,), mv[j], jnp.float32)
                    rows[k, pl.ds(0, LANES)] = rows[k, pl.ds(0, LANES)] * mj
                    rows[k, pl.ds(LANES, LANES)] = rows[k, pl.ds(LANES, LANES)] * mj

            for b in range(HSUP):
                pltpu.async_copy(
                    rows.at[pl.ds(half * HE + b * 128, 128), :],
                    s_sh.at[dstbuf.at[p, half * HSUP + b]], sem_s, add=True)

    pltpu.make_async_copy(out_ref.at[pl.ds(obase, HE), :],
                          rows.at[pl.ds(0, HE), :], semsa).wait()
    pltpu.make_async_copy(out_ref.at[pl.ds(obase, HE), :],
                          rows.at[pl.ds(HE, HE), :], semsb).wait()
    pltpu.make_async_copy(sidx_ref.at[c, pl.ds(row00, SUP), :],
                          idxbuf.at[0], semi).wait()
    pltpu.make_async_copy(dst_ref.at[pl.ds(row00, SUP), :],
                          dstbuf.at[0], semi).wait()
    pltpu.make_async_copy(mask_ref.at[pl.ds(row00 * 128, CHE)],
                          mbuf.at[0], semi).wait()
    plsc.subcore_barrier()
    for t in range(4):
        pltpu.sync_copy(s_sh.at[pl.ds(s * NPT + t * CHE, CHE), :], rows)
        pltpu.sync_copy(rows, out_ref.at[pl.ds(c * NPAD + s * NPT + t * CHE, CHE), :])
    tail = NPT - 4 * CHE
    pltpu.sync_copy(s_sh.at[pl.ds(s * NPT + 4 * CHE, tail), :],
                    rows.at[pl.ds(0, tail), :])
    pltpu.sync_copy(rows.at[pl.ds(0, tail), :],
                    out_ref.at[pl.ds(c * NPAD + s * NPT + 4 * CHE, tail), :])


@functools.cache
def _scatter_rows_call():
    mesh = plsc.VectorSubcoreMesh(core_axis_name="c", subcore_axis_name="s")
    return pl.kernel(
        _scatter_rows_body,
        out_type=jax.ShapeDtypeStruct((2 * NPAD, H), jnp.float32),
        mesh=mesh,
        scratch_types=[
            pltpu.VMEM((2, SUP, 128), jnp.int32),
            pltpu.VMEM((2, SUP, 128), jnp.int32),
            pltpu.VMEM((2, CHE), jnp.float32),
            pltpu.VMEM((CHE, H), jnp.float32),
            pltpu.VMEM_SHARED((NPAD, H), jnp.float32),
            pltpu.SemaphoreType.DMA,
            pltpu.SemaphoreType.DMA,
            pltpu.SemaphoreType.DMA,
            pltpu.SemaphoreType.DMA,
            pltpu.SemaphoreType.DMA,
        ],
        compiler_params=pltpu.CompilerParams(use_tc_tiling_on_sc=False),
    )


_ZPT = 2 * NPAD // NS


def _degwsum_body(dst_ref, dstw_ref, mask_ref, ones_ref, out_ref,
                  dstbuf, dstwbuf, mbuf, obuf, dw_sh, sems):
    c = lax.axis_index("c")
    s = lax.axis_index("s")
    wid = c * NS + s

    @pl.loop(0, CHE // LANES)
    def _zero(i):
        obuf[pl.ds(i * LANES, LANES)] = jnp.zeros((LANES,), jnp.float32)

    for t in range(8):
        pltpu.sync_copy(obuf, dw_sh.at[pl.ds(s * _ZPT + t * CHE, CHE)])
    pltpu.sync_copy(obuf.at[pl.ds(0, _ZPT - 8 * CHE)],
                    dw_sh.at[pl.ds(s * _ZPT + 8 * CHE, _ZPT - 8 * CHE)])
    plsc.subcore_barrier()

    @pl.loop(0, SUPW)
    def _chunk(ch):
        row0 = wid * RPW + ch * SUP
        pltpu.sync_copy(dst_ref.at[pl.ds(row0, SUP), :], dstbuf)
        pltpu.sync_copy(dstw_ref.at[pl.ds(row0, SUP), :], dstwbuf)
        pltpu.sync_copy(mask_ref.at[pl.ds(row0 * 128, SUP * 128)], mbuf)
        pltpu.sync_copy(ones_ref.at[pl.ds(row0 * 128, SUP * 128)], obuf)
        copies = []
        for b in range(SUP):
            copies.append(pltpu.async_copy(
                obuf.at[pl.ds(b * 128, 128)], dw_sh.at[dstbuf.at[b]],
                sems, add=True))
            copies.append(pltpu.async_copy(
                mbuf.at[pl.ds(b * 128, 128)], dw_sh.at[dstwbuf.at[b]],
                sems, add=True))
        for cd in copies:
            cd.wait()

    plsc.subcore_barrier()
    for t in range(8):
        pltpu.sync_copy(dw_sh.at[pl.ds(s * _ZPT + t * CHE, CHE)], obuf)
        pltpu.sync_copy(obuf, out_ref.at[pl.ds(c * 2 * NPAD + s * _ZPT + t * CHE, CHE)])
    tail = _ZPT - 8 * CHE
    pltpu.sync_copy(dw_sh.at[pl.ds(s * _ZPT + 8 * CHE, tail)], obuf.at[pl.ds(0, tail)])
    pltpu.sync_copy(obuf.at[pl.ds(0, tail)],
                    out_ref.at[pl.ds(c * 2 * NPAD + s * _ZPT + 8 * CHE, tail)])


@functools.cache
def _degwsum_call():
    mesh = plsc.VectorSubcoreMesh(core_axis_name="c", subcore_axis_name="s")
    return pl.kernel(
        _degwsum_body,
        out_type=jax.ShapeDtypeStruct((4 * NPAD,), jnp.float32),
        mesh=mesh,
        scratch_types=[
            pltpu.VMEM((SUP, 128), jnp.int32),
            pltpu.VMEM((SUP, 128), jnp.int32),
            pltpu.VMEM((SUP * 128,), jnp.float32),
            pltpu.VMEM((SUP * 128,), jnp.float32),
            pltpu.VMEM_SHARED((2 * NPAD,), jnp.float32),
            pltpu.SemaphoreType.DMA,
        ],
        compiler_params=pltpu.CompilerParams(use_tc_tiling_on_sc=False),
    )


def kernel(v, l, e, e_mask, vbi, vfc, vfb, W_feat, b_feat, W_lab, b_lab,
           W_msg, b_msg, W_upd, b_upd):
    f32 = jnp.float32
    i32 = jnp.int32

    src = e[0].astype(i32)
    dst = e[1].astype(i32)
    pad = EPAD - E
    srcp = jnp.concatenate([src, jnp.zeros((pad,), i32)])
    dstp = jnp.concatenate([dst, jnp.zeros((pad,), i32)])
    maskp = jnp.concatenate([e_mask.astype(f32), jnp.zeros((pad,), f32)])
    sidx = jnp.stack([srcp, srcp + NPAD]).reshape(2, NR, 128)
    dst2 = dstp.reshape(NR, 128)
    dstw2 = (dstp + NPAD).reshape(NR, 128)
    onesp = jnp.concatenate([jnp.ones((E,), f32), jnp.zeros((pad,), f32)])

    npad = 51200 - N
    vfcp = jnp.concatenate([vfc.astype(i32), jnp.full((npad,), _BIGI, i32)]).reshape(400, 128)
    vbip = jnp.concatenate([vbi.astype(i32), jnp.full((npad,), -1, i32)]).reshape(400, 128)

    pe24 = jnp.asarray(_pe_table())
    wf8 = jnp.concatenate([W_feat, jnp.zeros((2, H), f32)], axis=0)
    bf8 = jnp.broadcast_to(b_feat, (8, H))
    bl8 = jnp.broadcast_to(b_lab, (8, H))
    wm1 = W_msg[:D]
    wm2 = W_msg[D:2 * D]
    wm3 = jnp.concatenate([W_msg[2 * D:], jnp.zeros((5, D), f32)], axis=0)
    bm8 = jnp.broadcast_to(b_msg, (8, D))
    wu1 = W_upd[:D]
    wu2 = W_upd[D:]
    bu8 = jnp.broadcast_to(b_upd, (8, D))

    pool = _pool_call()(vfcp, vbip)
    x, pw3, g = _encode_call()(
        v, l, vfc.reshape(N, 1).astype(i32), vbi.reshape(N, 1).astype(i32),
        pool, pe24, wf8, bf8, W_lab, bl8, wm3, wm2)
    dw = jnp.transpose(_degwsum_call()(dst2, dstw2, maskp, onesp).reshape(4, NPAD))

    upd = _update_call(True)
    upd_last = _update_call(False)
    for layer in range(3):
        s2 = _scatter_rows_call()(sidx, dst2, maskp, g.reshape(2 * NPAD, H))
        s3 = s2.reshape(2, NPAD, H)
        if layer < 2:
            x, g = upd(x, s3, pw3, dw, wm1, bm8, wu1, wu2, bu8, wm2)
        else:
            (x,) = upd_last(x, s3, pw3, dw, wm1, bm8, wu1, wu2, bu8, wm2)
    return x


# speedup vs baseline: 9.5973x; 1.0217x over previous
import functools

import jax
import jax.numpy as jnp
import numpy as np
from jax import lax
from jax.experimental import pallas as pl
from jax.experimental.pallas import tpu as pltpu
from jax.experimental.pallas import tpu_sc as plsc

N = 50000
E = 800000
H = 32
D = 64
B = 16

NC = 2
NS = 16
LANES = 16

NPAD = 50048
EPAD = 811008
NR = EPAD // 128
RPT = NR // NS
RPW = NR // (NC * NS)
SUP = 6
HSUP = SUP // 2
CHE = SUP * 128
HE = CHE // 2
NSUP = RPT // SUP
SUPW = RPW // SUP
NPT = NPAD // NS

BN = 2000
NB = N // BN


def _pe_table() -> np.ndarray:
    pe = np.zeros((20, H), dtype=np.float32)
    position = np.arange(0, 20, dtype=np.float32)[:, None]
    div_term = np.exp(np.arange(0, H, 2, dtype=np.float32) * (-np.log(10000.0) / H))
    pe[:, 0::2] = np.sin(position * div_term)
    pe[:, 1::2] = np.cos(position * div_term)
    return np.concatenate([pe, np.zeros((4, H), np.float32)], axis=0)


_BIGI = np.int32(2**30)


def _pool_body(vfc_ref, vbi_ref, out_ref):
    vfc = vfc_ref[...]
    vbi = vbi_ref[...]
    acc = jnp.zeros((B, 128), jnp.float32)
    sub = lax.broadcasted_iota(jnp.int32, (B, 128), 0)
    for b in range(B):
        mb = jnp.min(jnp.where(vbi == b, vfc, _BIGI))
        acc = jnp.where(sub == b, mb.astype(jnp.float32), acc)
    out_ref[...] = acc


def _encode_body(v_ref, l_ref, vfc_ref, vbi_ref, pool_ref, pe_ref, wf_ref,
                 bf_ref, wl_ref, bl_ref, wm3_ref, wm2_ref,
                 x_ref, pw3_ref, g_ref):
    f32 = jnp.float32
    vb = v_ref[...]
    z2 = jnp.zeros((BN, 2), f32)
    nonpos = jnp.concatenate([vb[:, 0:3], vb[:, 6:9], z2], axis=-1)
    pos8 = jnp.concatenate([vb[:, 3:6], jnp.zeros((BN, 5), f32)], axis=-1)
    h = jnp.dot(nonpos, wf_ref[...], precision="highest") + bf_ref[0:1, :]
    pw3 = jnp.dot(pos8, wm3_ref[...], precision="highest")
    vbi = vbi_ref[...]
    vfc = vfc_ref[...]
    oh16 = (vbi == lax.broadcasted_iota(jnp.int32, (BN, B), 1)).astype(f32)
    poolg = jnp.dot(oh16, pool_ref[...], precision="highest")[:, 0:1]
    lvl = vfc - poolg.astype(jnp.int32)
    oh24 = (lvl == lax.broadcasted_iota(jnp.int32, (BN, 24), 1)).astype(f32)
    pe_emb = jnp.dot(oh24, pe_ref[...], precision="highest")
    le = jnp.dot(l_ref[...], wl_ref[...], precision="highest") + bl_ref[0:1, :]
    x = jnp.concatenate([h + pe_emb, le], axis=-1)
    x_ref[...] = x
    pw3_ref[...] = pw3
    g = jnp.dot(x, wm2_ref[...], precision="highest") - pw3
    g_ref[0] = g[:, :H]
    g_ref[1] = g[:, H:]


def _update_body(emit_g, x_ref, s_ref, pw3_ref, dw_ref, wa_ref, bm_ref,
                 wu2_ref, bu_ref, wm2_ref, xo_ref, *maybe_g):
    xb = x_ref[...]
    pw3 = pw3_ref[...]
    sb = jnp.concatenate([s_ref[0], s_ref[1]], axis=-1)
    dwb = dw_ref[...]
    deg = jnp.maximum(dwb[:, 0:1] + dwb[:, 2:3], 1.0)
    wsum = dwb[:, 1:2] + dwb[:, 3:4]
    t = jnp.dot(xb, wa_ref[...], precision="highest")
    base = t[:, :D] + pw3 + bm_ref[0:1, :]
    aggr = (base * wsum + sb) / deg
    xn = xb + t[:, D:] + jnp.dot(aggr, wu2_ref[...], precision="highest") + bu_ref[0:1, :]
    xo_ref[...] = xn
    if emit_g:
        g_ref = maybe_g[0]
        g = jnp.dot(xn, wm2_ref[...], precision="highest") - pw3
        g_ref[0] = g[:, :H]
        g_ref[1] = g[:, H:]


def _row_spec(w):
    return pl.BlockSpec((BN, w), lambda i: (i, 0))


def _full_spec(shape):
    nd = len(shape)
    return pl.BlockSpec(shape, lambda i, _n=nd: (0,) * _n)


def _split_spec(w):
    return pl.BlockSpec((2, BN, w), lambda i: (0, i, 0))


@functools.cache
def _pool_call():
    return pl.pallas_call(
        _pool_body,
        out_shape=jax.ShapeDtypeStruct((B, 128), jnp.float32),
    )


@functools.cache
def _encode_call():
    return pl.pallas_call(
        _encode_body,
        grid=(NB,),
        in_specs=[
            _row_spec(9), _row_spec(8), _row_spec(1), _row_spec(1),
            _full_spec((B, 128)), _full_spec((24, H)), _full_spec((8, H)),
            _full_spec((8, H)), _full_spec((8, H)), _full_spec((8, H)),
            _full_spec((8, D)), _full_spec((D, D)),
        ],
        out_specs=[_row_spec(D), _row_spec(D), _split_spec(H)],
        out_shape=[
            jax.ShapeDtypeStruct((N, D), jnp.float32),
            jax.ShapeDtypeStruct((N, D), jnp.float32),
            jax.ShapeDtypeStruct((2, NPAD, H), jnp.float32),
        ],
    )


@functools.cache
def _update_call(emit_g):
    out_specs = [_row_spec(D)]
    out_shape = [jax.ShapeDtypeStruct((N, D), jnp.float32)]
    if emit_g:
        out_specs.append(_split_spec(H))
        out_shape.append(jax.ShapeDtypeStruct((2, NPAD, H), jnp.float32))
    return pl.pallas_call(
        functools.partial(_update_body, emit_g),
        grid=(NB,),
        in_specs=[
            _row_spec(D), _split_spec(H), _row_spec(D),
            pl.BlockSpec((BN, 4), lambda i: (i, 0)),
            _full_spec((D, 2 * D)), _full_spec((8, D)),
            _full_spec((D, D)), _full_spec((8, D)), _full_spec((D, D)),
        ],
        out_specs=out_specs,
        out_shape=out_shape,
    )


def _scatter_rows_body(sidx_ref, dst_ref, mask_ref, g2_ref, out_ref,
                       idxbuf, dstbuf, mbuf, rows, s_sh,
                       semi, semga, semgb, semsa, semsb):
    c = lax.axis_index("c")
    s = lax.axis_index("s")

    @pl.loop(0, CHE)
    def _zero(i):
        z = jnp.zeros((LANES,), jnp.float32)
        rows[i, pl.ds(0, LANES)] = z
        rows[i, pl.ds(LANES, LANES)] = z

    for t in range(4):
        pltpu.sync_copy(rows, s_sh.at[pl.ds(s * NPT + t * CHE, CHE), :])
    pltpu.sync_copy(rows.at[pl.ds(0, NPT - 4 * CHE), :],
                    s_sh.at[pl.ds(s * NPT + 4 * CHE, NPT - 4 * CHE), :])

    obase = c * NPAD + s * NPT
    pltpu.async_copy(rows.at[pl.ds(0, HE), :], out_ref.at[pl.ds(obase, HE), :], semsa)
    pltpu.async_copy(rows.at[pl.ds(HE, HE), :], out_ref.at[pl.ds(obase + HE, HE), :], semsb)
    row00 = s * RPT
    pltpu.async_copy(sidx_ref.at[c, pl.ds(row00, SUP), :], idxbuf.at[0], semi)
    pltpu.async_copy(dst_ref.at[pl.ds(row00, SUP), :], dstbuf.at[0], semi)
    pltpu.async_copy(mask_ref.at[pl.ds(row00 * 128, CHE)], mbuf.at[0], semi)
    plsc.subcore_barrier()

    @pl.loop(0, NSUP)
    def _super(sup):
        p = lax.rem(sup, 2)
        q = 1 - p
        pltpu.make_async_copy(out_ref.at[pl.ds(obase, HE), :],
                              rows.at[pl.ds(0, HE), :], semsa).wait()
        pltpu.make_async_copy(out_ref.at[pl.ds(obase, HE), :],
                              rows.at[pl.ds(HE, HE), :], semsb).wait()
        pltpu.make_async_copy(sidx_ref.at[c, pl.ds(row00, SUP), :],
                              idxbuf.at[p], semi).wait()
        pltpu.make_async_copy(dst_ref.at[pl.ds(row00, SUP), :],
                              dstbuf.at[p], semi).wait()
        pltpu.make_async_copy(mask_ref.at[pl.ds(row00 * 128, CHE)],
                              mbuf.at[p], semi).wait()
        nrow0 = s * RPT + jnp.minimum(sup + 1, NSUP - 1) * SUP
        pltpu.async_copy(sidx_ref.at[c, pl.ds(nrow0, SUP), :], idxbuf.at[q], semi)
        pltpu.async_copy(dst_ref.at[pl.ds(nrow0, SUP), :], dstbuf.at[q], semi)
        pltpu.async_copy(mask_ref.at[pl.ds(nrow0 * 128, CHE)], mbuf.at[q], semi)
        ga, gb = [], []
        for b in range(HSUP):
            ga.append(pltpu.async_copy(
                g2_ref.at[idxbuf.at[p, b]],
                rows.at[pl.ds(b * 128, 128), :], semga))
        for b in range(HSUP):
            gb.append(pltpu.async_copy(
                g2_ref.at[idxbuf.at[p, HSUP + b]],
                rows.at[pl.ds(HE + b * 128, 128), :], semgb))
        for half, gds, sem_s in ((0, ga, semsa), (1, gb, semsb)):
            for gd in gds:
                gd.wait()

            @pl.loop(0, HE // LANES)
            def _group(gi):
                mv = mbuf[p, pl.ds(half * HE + gi * LANES, LANES)]
                for j in range(LANES):
                    k = half * HE + gi * LANES + j
                    mj = lax.gather(
                        mv, jnp.full((LANES, 1), j, jnp.int32),
                        lax.GatherDimensionNumbers(
                            offset_dims=(), collapsed_slice_dims=(0,),
                            start_index_map=(0,)),
                        (1,), mode=lax.GatherScatterMode.PROMISE_IN_BOUNDS)
                    rows[k, pl.ds(0, LANES)] = rows[k, pl.ds(0, LANES)] * mj
                    rows[k, pl.ds(LANES, LANES)] = rows[k, pl.ds(LANES, LANES)] * mj

            for b in range(HSUP):
                pltpu.async_copy(
                    rows.at[pl.ds(half * HE + b * 128, 128), :],
                    s_sh.at[dstbuf.at[p, half * HSUP + b]], sem_s, add=True)

    pltpu.make_async_copy(out_ref.at[pl.ds(obase, HE), :],
                          rows.at[pl.ds(0, HE), :], semsa).wait()
    pltpu.make_async_copy(out_ref.at[pl.ds(obase, HE), :],
                          rows.at[pl.ds(HE, HE), :], semsb).wait()
    pltpu.make_async_copy(sidx_ref.at[c, pl.ds(row00, SUP), :],
                          idxbuf.at[0], semi).wait()
    pltpu.make_async_copy(dst_ref.at[pl.ds(row00, SUP), :],
                          dstbuf.at[0], semi).wait()
    pltpu.make_async_copy(mask_ref.at[pl.ds(row00 * 128, CHE)],
                          mbuf.at[0], semi).wait()
    plsc.subcore_barrier()
    for t in range(4):
        pltpu.sync_copy(s_sh.at[pl.ds(s * NPT + t * CHE, CHE), :], rows)
        pltpu.sync_copy(rows, out_ref.at[pl.ds(c * NPAD + s * NPT + t * CHE, CHE), :])
    tail = NPT - 4 * CHE
    pltpu.sync_copy(s_sh.at[pl.ds(s * NPT + 4 * CHE, tail), :],
                    rows.at[pl.ds(0, tail), :])
    pltpu.sync_copy(rows.at[pl.ds(0, tail), :],
                    out_ref.at[pl.ds(c * NPAD + s * NPT + 4 * CHE, tail), :])


@functools.cache
def _scatter_rows_call():
    mesh = plsc.VectorSubcoreMesh(core_axis_name="c", subcore_axis_name="s")
    return pl.kernel(
        _scatter_rows_body,
        out_type=jax.ShapeDtypeStruct((2 * NPAD, H), jnp.float32),
        mesh=mesh,
        scratch_types=[
            pltpu.VMEM((2, SUP, 128), jnp.int32),
            pltpu.VMEM((2, SUP, 128), jnp.int32),
            pltpu.VMEM((2, CHE), jnp.float32),
            pltpu.VMEM((CHE, H), jnp.float32),
            pltpu.VMEM_SHARED((NPAD, H), jnp.float32),
            pltpu.SemaphoreType.DMA,
            pltpu.SemaphoreType.DMA,
            pltpu.SemaphoreType.DMA,
            pltpu.SemaphoreType.DMA,
            pltpu.SemaphoreType.DMA,
        ],
        compiler_params=pltpu.CompilerParams(use_tc_tiling_on_sc=False),
    )


_ZPT = 2 * NPAD // NS


def _degwsum_body(dst_ref, dstw_ref, mask_ref, ones_ref, out_ref,
                  dstbuf, dstwbuf, mbuf, obuf, dw_sh, sems):
    c = lax.axis_index("c")
    s = lax.axis_index("s")
    wid = c * NS + s

    @pl.loop(0, CHE // LANES)
    def _zero(i):
        obuf[pl.ds(i * LANES, LANES)] = jnp.zeros((LANES,), jnp.float32)

    for t in range(8):
        pltpu.sync_copy(obuf, dw_sh.at[pl.ds(s * _ZPT + t * CHE, CHE)])
    pltpu.sync_copy(obuf.at[pl.ds(0, _ZPT - 8 * CHE)],
                    dw_sh.at[pl.ds(s * _ZPT + 8 * CHE, _ZPT - 8 * CHE)])
    plsc.subcore_barrier()

    @pl.loop(0, SUPW)
    def _chunk(ch):
        row0 = wid * RPW + ch * SUP
        pltpu.sync_copy(dst_ref.at[pl.ds(row0, SUP), :], dstbuf)
        pltpu.sync_copy(dstw_ref.at[pl.ds(row0, SUP), :], dstwbuf)
        pltpu.sync_copy(mask_ref.at[pl.ds(row0 * 128, SUP * 128)], mbuf)
        pltpu.sync_copy(ones_ref.at[pl.ds(row0 * 128, SUP * 128)], obuf)
        copies = []
        for b in range(SUP):
            copies.append(pltpu.async_copy(
                obuf.at[pl.ds(b * 128, 128)], dw_sh.at[dstbuf.at[b]],
                sems, add=True))
            copies.append(pltpu.async_copy(
                mbuf.at[pl.ds(b * 128, 128)], dw_sh.at[dstwbuf.at[b]],
                sems, add=True))
        for cd in copies:
            cd.wait()

    plsc.subcore_barrier()
    for t in range(8):
        pltpu.sync_copy(dw_sh.at[pl.ds(s * _ZPT + t * CHE, CHE)], obuf)
        pltpu.sync_copy(obuf, out_ref.at[pl.ds(c * 2 * NPAD + s * _ZPT + t * CHE, CHE)])
    tail = _ZPT - 8 * CHE
    pltpu.sync_copy(dw_sh.at[pl.ds(s * _ZPT + 8 * CHE, tail)], obuf.at[pl.ds(0, tail)])
    pltpu.sync_copy(obuf.at[pl.ds(0, tail)],
                    out_ref.at[pl.ds(c * 2 * NPAD + s * _ZPT + 8 * CHE, tail)])


@functools.cache
def _degwsum_call():
    mesh = plsc.VectorSubcoreMesh(core_axis_name="c", subcore_axis_name="s")
    return pl.kernel(
        _degwsum_body,
        out_type=jax.ShapeDtypeStruct((4 * NPAD,), jnp.float32),
        mesh=mesh,
        scratch_types=[
            pltpu.VMEM((SUP, 128), jnp.int32),
            pltpu.VMEM((SUP, 128), jnp.int32),
            pltpu.VMEM((SUP * 128,), jnp.float32),
            pltpu.VMEM((SUP * 128,), jnp.float32),
            pltpu.VMEM_SHARED((2 * NPAD,), jnp.float32),
            pltpu.SemaphoreType.DMA,
        ],
        compiler_params=pltpu.CompilerParams(use_tc_tiling_on_sc=False),
    )


def kernel(v, l, e, e_mask, vbi, vfc, vfb, W_feat, b_feat, W_lab, b_lab,
           W_msg, b_msg, W_upd, b_upd):
    f32 = jnp.float32
    i32 = jnp.int32

    src = e[0].astype(i32)
    dst = e[1].astype(i32)
    pad = EPAD - E
    srcp = jnp.concatenate([src, jnp.zeros((pad,), i32)])
    dstp = jnp.concatenate([dst, jnp.zeros((pad,), i32)])
    maskp = jnp.concatenate([e_mask.astype(f32), jnp.zeros((pad,), f32)])
    sidx = jnp.stack([srcp, srcp + NPAD]).reshape(2, NR, 128)
    dst2 = dstp.reshape(NR, 128)
    dstw2 = (dstp + NPAD).reshape(NR, 128)
    onesp = jnp.concatenate([jnp.ones((E,), f32), jnp.zeros((pad,), f32)])

    npad = 51200 - N
    vfcp = jnp.concatenate([vfc.astype(i32), jnp.full((npad,), _BIGI, i32)]).reshape(400, 128)
    vbip = jnp.concatenate([vbi.astype(i32), jnp.full((npad,), -1, i32)]).reshape(400, 128)

    pe24 = jnp.asarray(_pe_table())
    wf8 = jnp.concatenate([W_feat, jnp.zeros((2, H), f32)], axis=0)
    bf8 = jnp.broadcast_to(b_feat, (8, H))
    bl8 = jnp.broadcast_to(b_lab, (8, H))
    wm1 = W_msg[:D]
    wa = jnp.concatenate([W_msg[:D], W_upd[:D]], axis=1)
    wm2 = W_msg[D:2 * D]
    wm3 = jnp.concatenate([W_msg[2 * D:], jnp.zeros((5, D), f32)], axis=0)
    bm8 = jnp.broadcast_to(b_msg, (8, D))
    wu1 = W_upd[:D]
    wu2 = W_upd[D:]
    bu8 = jnp.broadcast_to(b_upd, (8, D))

    pool = _pool_call()(vfcp, vbip)
    x, pw3, g = _encode_call()(
        v, l, vfc.reshape(N, 1).astype(i32), vbi.reshape(N, 1).astype(i32),
        pool, pe24, wf8, bf8, W_lab, bl8, wm3, wm2)
    dw = jnp.transpose(_degwsum_call()(dst2, dstw2, maskp, onesp).reshape(4, NPAD))

    upd = _update_call(True)
    upd_last = _update_call(False)
    for layer in range(3):
        s2 = _scatter_rows_call()(sidx, dst2, maskp, g.reshape(2 * NPAD, H))
        s3 = s2.reshape(2, NPAD, H)
        if layer < 2:
            x, g = upd(x, s3, pw3, dw, wa, bm8, wu2, bu8, wm2)
        else:
            (x,) = upd_last(x, s3, pw3, dw, wa, bm8, wu2, bu8, wm2)
    return x
